# baseline (speedup 1.0000x reference)
import jax
import jax.numpy as jnp
from jax import lax
from jax.experimental import pallas as pl
from jax.experimental.pallas import tpu as pltpu
from jax.experimental.pallas import tpu_sc as plsc

N = 10000
F = 128
NC = 2
NS = 16
NW = NC * NS
ROWS_PER_TILE = 632
N_PAD = NS * ROWS_PER_TILE
CH = 96
K = 105
E_PAD = NW * K * CH


def _segsum_sc(panels, srcb, dstb, zeros, ones_rows=None):
    nh = len(panels)
    with_deg = ones_rows is not None
    mesh = plsc.VectorSubcoreMesh(core_axis_name="c", subcore_axis_name="s")
    n_out = nh + (1 if with_deg else 0)
    out_type = [jax.ShapeDtypeStruct((NC, N_PAD, F), jnp.float32)
                for _ in range(n_out)]
    scratch = [
        pltpu.VMEM((K * CH,), jnp.int32),
        pltpu.VMEM((CH,), jnp.int32),
        pltpu.VMEM((CH,), jnp.int32),
        pltpu.VMEM((CH,), jnp.int32),
        pltpu.VMEM((CH, F), jnp.float32),
        pltpu.VMEM((CH, F), jnp.float32),
        pltpu.VMEM((CH, F), jnp.float32),
        pltpu.SemaphoreType.DMA,
        pltpu.SemaphoreType.DMA,
        pltpu.SemaphoreType.DMA,
        pltpu.SemaphoreType.DMA,
        pltpu.SemaphoreType.DMA,
        pltpu.SemaphoreType.DMA,
        pltpu.VMEM_SHARED((N_PAD, F), jnp.float32),
    ]

    def body(*refs):
        i = 0
        panel_r = refs[i:i + nh]; i += nh
        srcb_r, dstb_r, zeros_r = refs[i:i + 3]; i += 3
        if with_deg:
            ones_r = refs[i]; i += 1
        agg_out = refs[i:i + n_out]; i += n_out
        src_all = refs[i]; i += 1
        dstv = refs[i:i + 3]; i += 3
        rows = refs[i:i + 3]; i += 3
        semG = refs[i:i + 3]; i += 3
        semD = refs[i:i + 3]; i += 3
        agg_sp = refs[i]

        c = lax.axis_index("c")
        s = lax.axis_index("s")
        wid = s * NC + c
        r0 = s * ROWS_PER_TILE

        pltpu.sync_copy(srcb_r.at[wid], src_all)

        def run_pass(ph, out_ref, gather):
            pltpu.sync_copy(zeros_r.at[pl.ds(r0, ROWS_PER_TILE)],
                            agg_sp.at[pl.ds(r0, ROWS_PER_TILE)])
            plsc.subcore_barrier()

            for t in range(2):
                if gather:
                    pltpu.async_copy(ph.at[src_all.at[pl.ds(t * CH, CH)]],
                                     rows[t], semG[t])
                pltpu.async_copy(dstb_r.at[wid, t], dstv[t], semD[t])

            @pl.loop(0, K, step=3)
            def _(j):
                for t in range(3):
                    u = (t + 2) % 3

                    @pl.when(j + t + 2 < K)
                    def _():
                        if gather:
                            pltpu.async_copy(
                                ph.at[src_all.at[pl.ds((j + t + 2) * CH, CH)]],
                                rows[u], semG[u])
                        pltpu.async_copy(dstb_r.at[wid, j + t + 2],
                                         dstv[u], semD[u])
                    pltpu.make_async_copy(dstb_r.at[wid, j + t],
                                          dstv[t], semD[t]).wait()
                    if gather:
                        pltpu.make_async_copy(
                            ph.at[src_all.at[pl.ds((j + t) * CH, CH)]],
                            rows[t], semG[t]).wait()
                        pltpu.sync_copy(rows[t], agg_sp.at[dstv[t]], add=True)
                    else:
                        pltpu.sync_copy(rows[0], agg_sp.at[dstv[t]], add=True)

            plsc.subcore_barrier()
            pltpu.sync_copy(agg_sp.at[pl.ds(r0, ROWS_PER_TILE)],
                            out_ref.at[c, pl.ds(r0, ROWS_PER_TILE)])
            plsc.subcore_barrier()

        for h in range(nh):
            run_pass(panel_r[h], agg_out[h], True)
        if with_deg:
            pltpu.sync_copy(ones_r, rows[0])
            run_pass(None, agg_out[nh], False)

    args = list(panels) + [srcb, dstb, zeros]
    if with_deg:
        args.append(ones_rows)
    outs = pl.kernel(body, out_type=tuple(out_type), mesh=mesh,
                     scratch_types=tuple(scratch))(*args)
    if not isinstance(outs, (tuple, list)):
        outs = (outs,)
    return list(outs)


def _layer_tc(xhs, aggs, deg8, Wl, Wr, b, relu):
    nin = len(xhs)
    din = nin * F
    dout = Wl.shape[1]
    nout = dout // F
    BM = 400
    grid = (N // BM,)

    def body(*refs):
        xs = refs[:nin]
        ags = refs[nin:2 * nin]
        degr, wl, wr, br = refs[2 * nin:2 * nin + 4]
        outs = refs[2 * nin + 4:]
        deg = degr[...]
        dsum = deg[0, :, 0:1] + deg[1, :, 0:1]
        dinv = 1.0 / jnp.maximum(dsum, 1.0)
        acc = jnp.broadcast_to(br[...], (BM, dout)).astype(jnp.float32)
        for h in range(nin):
            a = ags[h][...]
            mean_h = (a[0] + a[1]) * dinv
            acc = acc + jnp.dot(mean_h, wl[pl.ds(h * F, F), :],
                                preferred_element_type=jnp.float32)
            acc = acc + jnp.dot(xs[h][...], wr[pl.ds(h * F, F), :],
                                preferred_element_type=jnp.float32)
        if relu:
            acc = jnp.maximum(acc, 0.0)
        for g in range(nout):
            outs[g][...] = acc[:, g * F:(g + 1) * F]

    in_specs = (
        [pl.BlockSpec((BM, F), lambda i: (i, 0)) for _ in range(nin)]
        + [pl.BlockSpec((NC, BM, F), lambda i: (0, i, 0)) for _ in range(nin)]
        + [pl.BlockSpec((NC, BM, F), lambda i: (0, i, 0)),
           pl.BlockSpec((din, dout), lambda i: (0, 0)),
           pl.BlockSpec((din, dout), lambda i: (0, 0)),
           pl.BlockSpec((1, dout), lambda i: (0, 0))]
    )
    out_specs = [pl.BlockSpec((BM, F), lambda i: (i, 0)) for _ in range(nout)]
    out_shape = [jax.ShapeDtypeStruct((N, F), jnp.float32) for _ in range(nout)]
    outs = pl.pallas_call(body, grid=grid, in_specs=in_specs,
                          out_specs=out_specs, out_shape=out_shape)(
        *xhs, *aggs, deg8, Wl, Wr, b)
    return list(outs)


def kernel(x, edge_index, Wl1, Wr1, b1, Wl2, Wr2, b2, Wl3, Wr3, b3, Wl4, Wr4, b4):
    ei = edge_index.astype(jnp.int32)
    src, dst = ei[0], ei[1]
    p = E_PAD - src.shape[0]
    pad = jnp.arange(p, dtype=jnp.int32)
    srcb = jnp.concatenate([src, pad % N]).reshape(NW, K * CH)
    dstb = jnp.concatenate([dst, N + pad % (N_PAD - N)]).reshape(NW, K, CH)
    zeros = jnp.zeros((N_PAD, F), jnp.float32)
    ones_rows = jnp.ones((CH, F), jnp.float32)

    a1, deg8 = _segsum_sc([x], srcb, dstb, zeros, ones_rows)
    a1 = [a1]
    h1 = _layer_tc([x], a1, deg8, Wl1, Wr1, b1.reshape(1, -1), True)
    a2 = _segsum_sc(h1, srcb, dstb, zeros)
    h2 = _layer_tc(h1, a2, deg8, Wl2, Wr2, b2.reshape(1, -1), True)
    a3 = _segsum_sc(h2, srcb, dstb, zeros)
    h3 = _layer_tc(h2, a3, deg8, Wl3, Wr3, b3.reshape(1, -1), True)
    a4 = _segsum_sc(h3, srcb, dstb, zeros)
    h4 = _layer_tc(h3, a4, deg8, Wl4, Wr4, b4.reshape(1, -1), False)
    return h4[0]


# speedup vs baseline: 1.0465x; 1.0465x over previous
import jax
import jax.numpy as jnp
from jax import lax
from jax.experimental import pallas as pl
from jax.experimental.pallas import tpu as pltpu
from jax.experimental.pallas import tpu_sc as plsc

N = 10000
F = 128
NC = 2
NS = 16
NW = NC * NS
ROWS_PER_TILE = 632
N_PAD = NS * ROWS_PER_TILE
CH = 96
K = 105
E_PAD = NW * K * CH


def _segsum_sc(panels, srcb, dstb, zeros, ones_rows=None):
    nh = len(panels)
    with_deg = ones_rows is not None
    mesh = plsc.VectorSubcoreMesh(core_axis_name="c", subcore_axis_name="s")
    n_out = nh + (1 if with_deg else 0)
    out_type = [jax.ShapeDtypeStruct((NC, N_PAD, F), jnp.float32)
                for _ in range(n_out)]
    scratch = [
        pltpu.VMEM((K * CH,), jnp.int32),
        pltpu.VMEM((CH,), jnp.int32),
        pltpu.VMEM((CH,), jnp.int32),
        pltpu.VMEM((CH,), jnp.int32),
        pltpu.VMEM((CH, F), jnp.float32),
        pltpu.VMEM((CH, F), jnp.float32),
        pltpu.VMEM((CH, F), jnp.float32),
        pltpu.SemaphoreType.DMA,
        pltpu.SemaphoreType.DMA,
        pltpu.SemaphoreType.DMA,
        pltpu.SemaphoreType.DMA,
        pltpu.SemaphoreType.DMA,
        pltpu.SemaphoreType.DMA,
        pltpu.VMEM_SHARED((N_PAD, F), jnp.float32),
    ]

    def body(*refs):
        i = 0
        panel_r = refs[i:i + nh]; i += nh
        srcb_r, dstb_r, zeros_r = refs[i:i + 3]; i += 3
        if with_deg:
            ones_r = refs[i]; i += 1
        agg_out = refs[i:i + n_out]; i += n_out
        src_all = refs[i]; i += 1
        dstv = refs[i:i + 3]; i += 3
        rows = refs[i:i + 3]; i += 3
        semG = refs[i:i + 3]; i += 3
        semD = refs[i:i + 3]; i += 3
        agg_sp = refs[i]

        c = lax.axis_index("c")
        s = lax.axis_index("s")
        wid = s * NC + c
        r0 = s * ROWS_PER_TILE

        pltpu.sync_copy(srcb_r.at[wid], src_all)

        def run_pass(ph, out_ref, gather):
            pltpu.sync_copy(zeros_r.at[pl.ds(r0, ROWS_PER_TILE)],
                            agg_sp.at[pl.ds(r0, ROWS_PER_TILE)])
            plsc.subcore_barrier()

            for t in range(2):
                if gather:
                    pltpu.async_copy(ph.at[src_all.at[pl.ds(t * CH, CH)]],
                                     rows[t], semG[t])
                pltpu.async_copy(dstb_r.at[wid, t], dstv[t], semD[t])

            @pl.loop(0, K, step=3)
            def _(j):
                for t in range(3):
                    u = (t + 2) % 3

                    @pl.when(j + t + 2 < K)
                    def _():
                        if gather:
                            pltpu.async_copy(
                                ph.at[src_all.at[pl.ds((j + t + 2) * CH, CH)]],
                                rows[u], semG[u])
                        pltpu.async_copy(dstb_r.at[wid, j + t + 2],
                                         dstv[u], semD[u])
                    pltpu.make_async_copy(dstb_r.at[wid, j + t],
                                          dstv[t], semD[t]).wait()
                    if gather:
                        pltpu.make_async_copy(
                            ph.at[src_all.at[pl.ds((j + t) * CH, CH)]],
                            rows[t], semG[t]).wait()
                        pltpu.sync_copy(rows[t], agg_sp.at[dstv[t]], add=True)
                    else:
                        pltpu.sync_copy(rows[0], agg_sp.at[dstv[t]], add=True)

            plsc.subcore_barrier()
            pltpu.sync_copy(agg_sp.at[pl.ds(r0, ROWS_PER_TILE)],
                            out_ref.at[c, pl.ds(r0, ROWS_PER_TILE)])
            plsc.subcore_barrier()

        for h in range(nh):
            run_pass(panel_r[h], agg_out[h], True)
        if with_deg:
            pltpu.sync_copy(ones_r, rows[0])
            run_pass(None, agg_out[nh], False)

    args = list(panels) + [srcb, dstb, zeros]
    if with_deg:
        args.append(ones_rows)
    outs = pl.kernel(body, out_type=tuple(out_type), mesh=mesh,
                     scratch_types=tuple(scratch))(*args)
    if not isinstance(outs, (tuple, list)):
        outs = (outs,)
    return list(outs)


def _layer_tc(xhs, aggs, deg8, Wl, Wr, b, relu):
    nin = len(xhs)
    din = nin * F
    dout = Wl.shape[1]
    nout = dout // F
    BM = 1000
    grid = (N // BM,)

    def body(*refs):
        xs = refs[:nin]
        ags = refs[nin:2 * nin]
        degr, wl, wr, br = refs[2 * nin:2 * nin + 4]
        outs = refs[2 * nin + 4:]
        deg = degr[...]
        dsum = deg[0, :, 0:1] + deg[1, :, 0:1]
        dinv = 1.0 / jnp.maximum(dsum, 1.0)
        acc = jnp.broadcast_to(br[...], (BM, dout)).astype(jnp.float32)
        for h in range(nin):
            a = ags[h][...]
            mean_h = (a[0] + a[1]) * dinv
            acc = acc + jnp.dot(mean_h, wl[pl.ds(h * F, F), :],
                                preferred_element_type=jnp.float32)
            acc = acc + jnp.dot(xs[h][...], wr[pl.ds(h * F, F), :],
                                preferred_element_type=jnp.float32)
        if relu:
            acc = jnp.maximum(acc, 0.0)
        for g in range(nout):
            outs[g][...] = acc[:, g * F:(g + 1) * F]

    in_specs = (
        [pl.BlockSpec((BM, F), lambda i: (i, 0)) for _ in range(nin)]
        + [pl.BlockSpec((NC, BM, F), lambda i: (0, i, 0)) for _ in range(nin)]
        + [pl.BlockSpec((NC, BM, F), lambda i: (0, i, 0)),
           pl.BlockSpec((din, dout), lambda i: (0, 0)),
           pl.BlockSpec((din, dout), lambda i: (0, 0)),
           pl.BlockSpec((1, dout), lambda i: (0, 0))]
    )
    out_specs = [pl.BlockSpec((BM, F), lambda i: (i, 0)) for _ in range(nout)]
    out_shape = [jax.ShapeDtypeStruct((N, F), jnp.float32) for _ in range(nout)]
    outs = pl.pallas_call(body, grid=grid, in_specs=in_specs,
                          out_specs=out_specs, out_shape=out_shape)(
        *xhs, *aggs, deg8, Wl, Wr, b)
    return list(outs)


def kernel(x, edge_index, Wl1, Wr1, b1, Wl2, Wr2, b2, Wl3, Wr3, b3, Wl4, Wr4, b4):
    ei = edge_index.astype(jnp.int32)
    src, dst = ei[0], ei[1]
    p = E_PAD - src.shape[0]
    pad = jnp.arange(p, dtype=jnp.int32)
    srcb = jnp.concatenate([src, pad % N]).reshape(NW, K * CH)
    dstb = jnp.concatenate([dst, N + pad % (N_PAD - N)]).reshape(NW, K, CH)
    zeros = jnp.zeros((N_PAD, F), jnp.float32)
    ones_rows = jnp.ones((CH, F), jnp.float32)

    a1, deg8 = _segsum_sc([x], srcb, dstb, zeros, ones_rows)
    a1 = [a1]
    h1 = _layer_tc([x], a1, deg8, Wl1, Wr1, b1.reshape(1, -1), True)
    a2 = _segsum_sc(h1, srcb, dstb, zeros)
    h2 = _layer_tc(h1, a2, deg8, Wl2, Wr2, b2.reshape(1, -1), True)
    a3 = _segsum_sc(h2, srcb, dstb, zeros)
    h3 = _layer_tc(h2, a3, deg8, Wl3, Wr3, b3.reshape(1, -1), True)
    a4 = _segsum_sc(h3, srcb, dstb, zeros)
    h4 = _layer_tc(h3, a4, deg8, Wl4, Wr4, b4.reshape(1, -1), False)
    return h4[0]


# speedup vs baseline: 1.0570x; 1.0100x over previous
import jax
import jax.numpy as jnp
from jax import lax
from jax.experimental import pallas as pl
from jax.experimental.pallas import tpu as pltpu
from jax.experimental.pallas import tpu_sc as plsc

N = 10000
F = 128
NC = 2
NS = 16
NW = NC * NS
ROWS_PER_TILE = 632
N_PAD = NS * ROWS_PER_TILE
CH = 96
K = 105
E_PAD = NW * K * CH


def _segsum_sc(panels, srcb, dstb, zeros, ones_rows=None):
    nh = len(panels)
    with_deg = ones_rows is not None
    mesh = plsc.VectorSubcoreMesh(core_axis_name="c", subcore_axis_name="s")
    n_out = nh + (1 if with_deg else 0)
    out_type = [jax.ShapeDtypeStruct((NC, N_PAD, F), jnp.float32)
                for _ in range(n_out)]
    scratch = [
        pltpu.VMEM((K * CH,), jnp.int32),
        pltpu.VMEM((CH,), jnp.int32),
        pltpu.VMEM((CH,), jnp.int32),
        pltpu.VMEM((CH,), jnp.int32),
        pltpu.VMEM((CH, F), jnp.float32),
        pltpu.VMEM((CH, F), jnp.float32),
        pltpu.VMEM((CH, F), jnp.float32),
        pltpu.SemaphoreType.DMA,
        pltpu.SemaphoreType.DMA,
        pltpu.SemaphoreType.DMA,
        pltpu.SemaphoreType.DMA,
        pltpu.SemaphoreType.DMA,
        pltpu.SemaphoreType.DMA,
        pltpu.VMEM_SHARED((N_PAD, F), jnp.float32),
    ]

    def body(*refs):
        i = 0
        panel_r = refs[i:i + nh]; i += nh
        srcb_r, dstb_r, zeros_r = refs[i:i + 3]; i += 3
        if with_deg:
            ones_r = refs[i]; i += 1
        agg_out = refs[i:i + n_out]; i += n_out
        src_all = refs[i]; i += 1
        dstv = refs[i:i + 3]; i += 3
        rows = refs[i:i + 3]; i += 3
        semG = refs[i:i + 3]; i += 3
        semD = refs[i:i + 3]; i += 3
        agg_sp = refs[i]

        c = lax.axis_index("c")
        s = lax.axis_index("s")
        wid = s * NC + c
        r0 = s * ROWS_PER_TILE

        pltpu.sync_copy(srcb_r.at[wid], src_all)

        def run_pass(ph, out_ref, gather):
            pltpu.sync_copy(zeros_r.at[pl.ds(r0, ROWS_PER_TILE)],
                            agg_sp.at[pl.ds(r0, ROWS_PER_TILE)])
            plsc.subcore_barrier()

            for t in range(2):
                if gather:
                    pltpu.async_copy(ph.at[src_all.at[pl.ds(t * CH, CH)]],
                                     rows[t], semG[t])
                pltpu.async_copy(dstb_r.at[wid, t], dstv[t], semD[t])

            @pl.loop(0, K, step=3)
            def _(j):
                for t in range(3):
                    u = (t + 2) % 3

                    @pl.when(j + t + 2 < K)
                    def _():
                        if gather:
                            pltpu.async_copy(
                                ph.at[src_all.at[pl.ds((j + t + 2) * CH, CH)]],
                                rows[u], semG[u])
                        pltpu.async_copy(dstb_r.at[wid, j + t + 2],
                                         dstv[u], semD[u])
                    pltpu.make_async_copy(dstb_r.at[wid, j + t],
                                          dstv[t], semD[t]).wait()
                    if gather:
                        pltpu.make_async_copy(
                            ph.at[src_all.at[pl.ds((j + t) * CH, CH)]],
                            rows[t], semG[t]).wait()
                        pltpu.sync_copy(rows[t], agg_sp.at[dstv[t]], add=True)
                    else:
                        pltpu.sync_copy(rows[0], agg_sp.at[dstv[t]], add=True)

            plsc.subcore_barrier()
            pltpu.sync_copy(agg_sp.at[pl.ds(r0, ROWS_PER_TILE)],
                            out_ref.at[c, pl.ds(r0, ROWS_PER_TILE)])
            plsc.subcore_barrier()

        for h in range(nh):
            run_pass(panel_r[h], agg_out[h], True)
        if with_deg:
            pltpu.sync_copy(ones_r, rows[0])
            run_pass(None, agg_out[nh], False)

    args = list(panels) + [srcb, dstb, zeros]
    if with_deg:
        args.append(ones_rows)
    outs = pl.kernel(body, out_type=tuple(out_type), mesh=mesh,
                     scratch_types=tuple(scratch))(*args)
    if not isinstance(outs, (tuple, list)):
        outs = (outs,)
    return list(outs)


def _layer_tc(xhs, aggs, deg8, Wl, Wr, b, relu):
    nin = len(xhs)
    din = nin * F
    dout = Wl.shape[1]
    nout = dout // F
    BM = 2000
    grid = (N // BM,)

    def body(*refs):
        xs = refs[:nin]
        ags = refs[nin:2 * nin]
        degr, wl, wr, br = refs[2 * nin:2 * nin + 4]
        outs = refs[2 * nin + 4:]
        deg = degr[...]
        dsum = deg[0, :, 0:1] + deg[1, :, 0:1]
        dinv = 1.0 / jnp.maximum(dsum, 1.0)
        acc = jnp.broadcast_to(br[...], (BM, dout)).astype(jnp.float32)
        for h in range(nin):
            a = ags[h][...]
            mean_h = (a[0] + a[1]) * dinv
            acc = acc + jnp.dot(mean_h, wl[pl.ds(h * F, F), :],
                                preferred_element_type=jnp.float32)
            acc = acc + jnp.dot(xs[h][...], wr[pl.ds(h * F, F), :],
                                preferred_element_type=jnp.float32)
        if relu:
            acc = jnp.maximum(acc, 0.0)
        for g in range(nout):
            outs[g][...] = acc[:, g * F:(g + 1) * F]

    in_specs = (
        [pl.BlockSpec((BM, F), lambda i: (i, 0)) for _ in range(nin)]
        + [pl.BlockSpec((NC, BM, F), lambda i: (0, i, 0)) for _ in range(nin)]
        + [pl.BlockSpec((NC, BM, 8), lambda i: (0, i, 0)),
           pl.BlockSpec((din, dout), lambda i: (0, 0)),
           pl.BlockSpec((din, dout), lambda i: (0, 0)),
           pl.BlockSpec((1, dout), lambda i: (0, 0))]
    )
    out_specs = [pl.BlockSpec((BM, F), lambda i: (i, 0)) for _ in range(nout)]
    out_shape = [jax.ShapeDtypeStruct((N, F), jnp.float32) for _ in range(nout)]
    outs = pl.pallas_call(body, grid=grid, in_specs=in_specs,
                          out_specs=out_specs, out_shape=out_shape)(
        *xhs, *aggs, deg8, Wl, Wr, b)
    return list(outs)


def kernel(x, edge_index, Wl1, Wr1, b1, Wl2, Wr2, b2, Wl3, Wr3, b3, Wl4, Wr4, b4):
    ei = edge_index.astype(jnp.int32)
    src, dst = ei[0], ei[1]
    p = E_PAD - src.shape[0]
    pad = jnp.arange(p, dtype=jnp.int32)
    srcb = jnp.concatenate([src, pad % N]).reshape(NW, K * CH)
    dstb = jnp.concatenate([dst, N + pad % (N_PAD - N)]).reshape(NW, K, CH)
    zeros = jnp.zeros((N_PAD, F), jnp.float32)
    ones_rows = jnp.ones((CH, F), jnp.float32)

    a1, degp = _segsum_sc([x], srcb, dstb, zeros, ones_rows)
    a1 = [a1]
    deg8 = degp[:, :, :8]
    h1 = _layer_tc([x], a1, deg8, Wl1, Wr1, b1.reshape(1, -1), True)
    a2 = _segsum_sc(h1, srcb, dstb, zeros)
    h2 = _layer_tc(h1, a2, deg8, Wl2, Wr2, b2.reshape(1, -1), True)
    a3 = _segsum_sc(h2, srcb, dstb, zeros)
    h3 = _layer_tc(h2, a3, deg8, Wl3, Wr3, b3.reshape(1, -1), True)
    a4 = _segsum_sc(h3, srcb, dstb, zeros)
    h4 = _layer_tc(h3, a4, deg8, Wl4, Wr4, b4.reshape(1, -1), False)
    return h4[0]


# speedup vs baseline: 1.0732x; 1.0154x over previous
import jax
import jax.numpy as jnp
from jax import lax
from jax.experimental import pallas as pl
from jax.experimental.pallas import tpu as pltpu
from jax.experimental.pallas import tpu_sc as plsc

N = 10000
F = 128
NC = 2
NS = 16
NW = NC * NS
ROWS_PER_TILE = 632
N_PAD = NS * ROWS_PER_TILE
CH = 72
K = 140
E_PAD = NW * K * CH


def _segsum_sc(panels, srcb, dstb, zeros, ones_rows=None):
    nh = len(panels)
    with_deg = ones_rows is not None
    mesh = plsc.VectorSubcoreMesh(core_axis_name="c", subcore_axis_name="s")
    n_out = nh + (1 if with_deg else 0)
    out_type = [jax.ShapeDtypeStruct((NC, N_PAD, F), jnp.float32)
                for _ in range(n_out)]
    scratch = [
        pltpu.VMEM((K * CH,), jnp.int32),
        pltpu.VMEM((CH,), jnp.int32),
        pltpu.VMEM((CH,), jnp.int32),
        pltpu.VMEM((CH,), jnp.int32),
        pltpu.VMEM((CH,), jnp.int32),
        pltpu.VMEM((CH, F), jnp.float32),
        pltpu.VMEM((CH, F), jnp.float32),
        pltpu.VMEM((CH, F), jnp.float32),
        pltpu.VMEM((CH, F), jnp.float32),
        pltpu.SemaphoreType.DMA,
        pltpu.SemaphoreType.DMA,
        pltpu.SemaphoreType.DMA,
        pltpu.SemaphoreType.DMA,
        pltpu.SemaphoreType.DMA,
        pltpu.SemaphoreType.DMA,
        pltpu.SemaphoreType.DMA,
        pltpu.SemaphoreType.DMA,
        pltpu.VMEM_SHARED((N_PAD, F), jnp.float32),
    ]

    def body(*refs):
        i = 0
        panel_r = refs[i:i + nh]; i += nh
        srcb_r, dstb_r, zeros_r = refs[i:i + 3]; i += 3
        if with_deg:
            ones_r = refs[i]; i += 1
        agg_out = refs[i:i + n_out]; i += n_out
        src_all = refs[i]; i += 1
        dstv = refs[i:i + 4]; i += 4
        rows = refs[i:i + 4]; i += 4
        semG = refs[i:i + 4]; i += 4
        semD = refs[i:i + 4]; i += 4
        agg_sp = refs[i]

        c = lax.axis_index("c")
        s = lax.axis_index("s")
        wid = s * NC + c
        r0 = s * ROWS_PER_TILE

        pltpu.sync_copy(srcb_r.at[wid], src_all)

        def run_pass(ph, out_ref, gather):
            pltpu.sync_copy(zeros_r.at[pl.ds(r0, ROWS_PER_TILE)],
                            agg_sp.at[pl.ds(r0, ROWS_PER_TILE)])
            plsc.subcore_barrier()

            for t in range(3):
                if gather:
                    pltpu.async_copy(ph.at[src_all.at[pl.ds(t * CH, CH)]],
                                     rows[t], semG[t])
                pltpu.async_copy(dstb_r.at[wid, t], dstv[t], semD[t])

            @pl.loop(0, K, step=4)
            def _(j):
                for t in range(4):
                    u = (t + 3) % 4

                    @pl.when(j + t + 3 < K)
                    def _():
                        if gather:
                            pltpu.async_copy(
                                ph.at[src_all.at[pl.ds((j + t + 3) * CH, CH)]],
                                rows[u], semG[u])
                        pltpu.async_copy(dstb_r.at[wid, j + t + 3],
                                         dstv[u], semD[u])
                    pltpu.make_async_copy(dstb_r.at[wid, j + t],
                                          dstv[t], semD[t]).wait()
                    if gather:
                        pltpu.make_async_copy(
                            ph.at[src_all.at[pl.ds((j + t) * CH, CH)]],
                            rows[t], semG[t]).wait()
                        pltpu.sync_copy(rows[t], agg_sp.at[dstv[t]], add=True)
                    else:
                        pltpu.sync_copy(rows[0], agg_sp.at[dstv[t]], add=True)

            plsc.subcore_barrier()
            pltpu.sync_copy(agg_sp.at[pl.ds(r0, ROWS_PER_TILE)],
                            out_ref.at[c, pl.ds(r0, ROWS_PER_TILE)])
            plsc.subcore_barrier()

        for h in range(nh):
            run_pass(panel_r[h], agg_out[h], True)
        if with_deg:
            pltpu.sync_copy(ones_r, rows[0])
            run_pass(None, agg_out[nh], False)

    args = list(panels) + [srcb, dstb, zeros]
    if with_deg:
        args.append(ones_rows)
    outs = pl.kernel(body, out_type=tuple(out_type), mesh=mesh,
                     scratch_types=tuple(scratch))(*args)
    if not isinstance(outs, (tuple, list)):
        outs = (outs,)
    return list(outs)


def _layer_tc(xhs, aggs, deg8, Wl, Wr, b, relu):
    nin = len(xhs)
    din = nin * F
    dout = Wl.shape[1]
    nout = dout // F
    BM = 2000
    grid = (N // BM,)

    def body(*refs):
        xs = refs[:nin]
        ags = refs[nin:2 * nin]
        degr, wl, wr, br = refs[2 * nin:2 * nin + 4]
        outs = refs[2 * nin + 4:]
        deg = degr[...]
        dsum = deg[0, :, 0:1] + deg[1, :, 0:1]
        dinv = 1.0 / jnp.maximum(dsum, 1.0)
        acc = jnp.broadcast_to(br[...], (BM, dout)).astype(jnp.float32)
        for h in range(nin):
            a = ags[h][...]
            mean_h = (a[0] + a[1]) * dinv
            acc = acc + jnp.dot(mean_h, wl[pl.ds(h * F, F), :],
                                preferred_element_type=jnp.float32)
            acc = acc + jnp.dot(xs[h][...], wr[pl.ds(h * F, F), :],
                                preferred_element_type=jnp.float32)
        if relu:
            acc = jnp.maximum(acc, 0.0)
        for g in range(nout):
            outs[g][...] = acc[:, g * F:(g + 1) * F]

    in_specs = (
        [pl.BlockSpec((BM, F), lambda i: (i, 0)) for _ in range(nin)]
        + [pl.BlockSpec((NC, BM, F), lambda i: (0, i, 0)) for _ in range(nin)]
        + [pl.BlockSpec((NC, BM, 8), lambda i: (0, i, 0)),
           pl.BlockSpec((din, dout), lambda i: (0, 0)),
           pl.BlockSpec((din, dout), lambda i: (0, 0)),
           pl.BlockSpec((1, dout), lambda i: (0, 0))]
    )
    out_specs = [pl.BlockSpec((BM, F), lambda i: (i, 0)) for _ in range(nout)]
    out_shape = [jax.ShapeDtypeStruct((N, F), jnp.float32) for _ in range(nout)]
    outs = pl.pallas_call(body, grid=grid, in_specs=in_specs,
                          out_specs=out_specs, out_shape=out_shape)(
        *xhs, *aggs, deg8, Wl, Wr, b)
    return list(outs)


def kernel(x, edge_index, Wl1, Wr1, b1, Wl2, Wr2, b2, Wl3, Wr3, b3, Wl4, Wr4, b4):
    ei = edge_index.astype(jnp.int32)
    src, dst = ei[0], ei[1]
    p = E_PAD - src.shape[0]
    pad = jnp.arange(p, dtype=jnp.int32)
    srcb = jnp.concatenate([src, pad % N]).reshape(NW, K * CH)
    dstb = jnp.concatenate([dst, N + pad % (N_PAD - N)]).reshape(NW, K, CH)
    zeros = jnp.zeros((N_PAD, F), jnp.float32)
    ones_rows = jnp.ones((CH, F), jnp.float32)

    a1, degp = _segsum_sc([x], srcb, dstb, zeros, ones_rows)
    a1 = [a1]
    deg8 = degp[:, :, :8]
    h1 = _layer_tc([x], a1, deg8, Wl1, Wr1, b1.reshape(1, -1), True)
    a2 = _segsum_sc(h1, srcb, dstb, zeros)
    h2 = _layer_tc(h1, a2, deg8, Wl2, Wr2, b2.reshape(1, -1), True)
    a3 = _segsum_sc(h2, srcb, dstb, zeros)
    h3 = _layer_tc(h2, a3, deg8, Wl3, Wr3, b3.reshape(1, -1), True)
    a4 = _segsum_sc(h3, srcb, dstb, zeros)
    h4 = _layer_tc(h3, a4, deg8, Wl4, Wr4, b4.reshape(1, -1), False)
    return h4[0]
